# baseline (speedup 1.0000x reference)
import functools

import jax
import jax.numpy as jnp
import numpy as np
from jax import lax
from jax.experimental import pallas as pl
from jax.experimental.pallas import tpu as pltpu
from jax.experimental.pallas import tpu_sc as plsc

_VOCAB = 1000000
_DEPTH = 64
_LENGTH = 512
_BATCH = 1024

_LANES = 16
_VBLK = 128
_CH = 256


def _pos_encoding_t_np(length, depth):
    pos = np.arange(length)[:, None]
    i = np.arange(depth)[None, :]
    angle_rates = 1.0 / np.power(10000, 2 * (i // 2) / np.float32(depth))
    angle_rads = pos * angle_rates
    angle_rads[:, 0::2] = np.sin(angle_rads[:, 0::2])
    angle_rads[:, 1::2] = np.cos(angle_rads[:, 1::2])
    return np.ascontiguousarray(angle_rads.astype(np.float32).T)


def _make_relayout_kernel():
    info = plsc.get_sparse_core_info()
    nc, ns = info.num_cores, info.num_subcores
    nw = nc * ns
    n_blk = _VOCAB // _VBLK
    tail_v = _VOCAB - n_blk * _VBLK
    mesh = plsc.VectorSubcoreMesh(core_axis_name="c", subcore_axis_name="s")

    @functools.partial(
        pl.kernel,
        out_type=jax.ShapeDtypeStruct((_VOCAB // 2, 2 * _DEPTH), jnp.float32),
        mesh=mesh,
        scratch_types=[
            pltpu.VMEM((2, _DEPTH, _VBLK), jnp.float32),
            pltpu.VMEM((2, _DEPTH, _VBLK), jnp.float32),
            pltpu.VMEM((tail_v // 2, 2 * _DEPTH), jnp.float32),
            pltpu.SemaphoreType.DMA,
            pltpu.SemaphoreType.DMA,
            pltpu.SemaphoreType.DMA,
            pltpu.SemaphoreType.DMA,
        ],
        compiler_params=pltpu.CompilerParams(use_tc_tiling_on_sc=True, needs_layout_passes=False),
    )
    def k(tt_hbm, tail2_hbm, t2_hbm, slab_v, tout_v, tail_v_buf,
          in_sem0, in_sem1, out_sem0, out_sem1):
        wid = lax.axis_index("s") * nc + lax.axis_index("c")
        in_sems = [in_sem0, in_sem1]
        out_sems = [out_sem0, out_sem1]
        iota = lax.iota(jnp.int32, _LANES)
        half_iota = lax.shift_right_logical(iota, 1)
        parity64 = lax.shift_left(lax.bitwise_and(iota, 1), 6)

        n_mine = (n_blk - 1 - wid) // nw + 1

        def vb_of(i):
            return wid + i * nw

        def start_in(i, slot):
            vb = vb_of(i)
            return pltpu.async_copy(
                tt_hbm.at[:, pl.ds(vb * _VBLK, _VBLK)],
                slab_v.at[slot], in_sems[slot])

        def transpose_slab(slot):
            def body_vs(vs0, _):
                row16 = vs0 * 8 + half_iota

                def body_d(d, col16):
                    vals = slab_v[slot, d, pl.ds(vs0 * _LANES, _LANES)]
                    plsc.store_scatter(tout_v.at[slot], [row16, col16], vals)
                    return col16 + 1

                lax.fori_loop(0, _DEPTH, body_d, parity64, unroll=8)
                return 0

            lax.fori_loop(0, _VBLK // _LANES, body_vs, 0)

        @pl.when(n_mine > 0)
        def _():
            start_in(0, 0)

        def step(i, slot):
            @pl.when(i + 1 < n_mine)
            def _():
                start_in(i + 1, 1 - slot)

            pltpu.make_async_copy(
                tt_hbm.at[:, pl.ds(0, _VBLK)], slab_v.at[slot],
                in_sems[slot]).wait()

            @pl.when(i >= 2)
            def _():
                pltpu.make_async_copy(
                    tout_v.at[slot],
                    t2_hbm.at[pl.ds(0, _VBLK // 2)],
                    out_sems[slot]).wait()

            transpose_slab(slot)
            vb = vb_of(i)
            pltpu.async_copy(
                tout_v.at[slot],
                t2_hbm.at[pl.ds(vb * (_VBLK // 2), _VBLK // 2)],
                out_sems[slot])

        def pair_body(p, _):
            for sl in (0, 1):
                i = p * 2 + sl

                @pl.when(i < n_mine)
                def _():
                    step(i, sl)
            return 0

        lax.fori_loop(0, (n_mine + 1) // 2, pair_body, 0)

        for sl in (0, 1):
            @pl.when(n_mine > sl)
            def _():
                pltpu.make_async_copy(
                    tout_v.at[sl],
                    t2_hbm.at[pl.ds(0, _VBLK // 2)],
                    out_sems[sl]).wait()

        @pl.when(wid == 0)
        def _():
            pltpu.sync_copy(tail2_hbm, tail_v_buf)
            pltpu.sync_copy(
                tail_v_buf,
                t2_hbm.at[pl.ds(n_blk * (_VBLK // 2), tail_v // 2)])

    return k


def _make_gather_kernel():
    info = plsc.get_sparse_core_info()
    nc, ns = info.num_cores, info.num_subcores
    nw = nc * ns
    n_tok = _BATCH * _LENGTH
    per_w = n_tok // nw
    n_ch = per_w // _CH
    ch_per_seq = _LENGTH // _CH
    mesh = plsc.VectorSubcoreMesh(core_axis_name="c", subcore_axis_name="s")

    @functools.partial(
        pl.kernel,
        out_type=jax.ShapeDtypeStruct((_BATCH, _DEPTH, _LENGTH), jnp.float32),
        mesh=mesh,
        scratch_types=[
            pltpu.VMEM((_CH,), jnp.int32),
            pltpu.VMEM((_CH,), jnp.int32),
            pltpu.VMEM((_CH,), jnp.int32),
            pltpu.VMEM((_CH, 2 * _DEPTH), jnp.float32),
            pltpu.VMEM((_DEPTH, _CH), jnp.float32),
            pltpu.VMEM((_DEPTH * _LENGTH,), jnp.float32),
            pltpu.SemaphoreType.DMA,
        ],
        compiler_params=pltpu.CompilerParams(use_tc_tiling_on_sc=True, needs_layout_passes=False),
    )
    def k(x_hbm, t2_hbm, pe_hbm, out_hbm, idx_v, q_v, pcol_v, wide_v,
          outb_v, pe_v, sem):
        wid = lax.axis_index("s") * nc + lax.axis_index("c")
        base_w = wid * per_w
        pltpu.sync_copy(pe_hbm, pe_v)
        iota = lax.iota(jnp.int32, _LANES)

        def chunk_body(s, _):
            base = base_w + s * _CH
            b = lax.div(base, _LENGTH)
            half = lax.rem(s, ch_per_seq)
            l_off = half * _CH

            pltpu.sync_copy(x_hbm.at[pl.ds(base, _CH)], idx_v)

            def prep(i, _):
                v = idx_v[pl.ds(i * _LANES, _LANES)]
                q_v[pl.ds(i * _LANES, _LANES)] = lax.shift_right_logical(v, 1)
                pcol_v[pl.ds(i * _LANES, _LANES)] = lax.shift_left(
                    lax.bitwise_and(v, 1), 6)
                return 0

            lax.fori_loop(0, _CH // _LANES, prep, 0, unroll=4)

            pltpu.async_copy(t2_hbm.at[q_v], wide_v, sem).wait()

            def body_lb(lb, _):
                row16 = lb * _LANES + iota
                p16 = pcol_v[pl.ds(lb * _LANES, _LANES)]

                def body_d(d, col16):
                    g = plsc.load_gather(wide_v, [row16, col16])
                    pe16 = pe_v[pl.ds(d * _LENGTH + l_off + lb * _LANES,
                                      _LANES)]
                    outb_v[d, pl.ds(lb * _LANES, _LANES)] = g + pe16
                    return col16 + 1

                lax.fori_loop(0, _DEPTH, body_d, p16, unroll=8)
                return 0

            lax.fori_loop(0, _CH // _LANES, body_lb, 0)

            pltpu.sync_copy(outb_v, out_hbm.at[b, :, pl.ds(l_off, _CH)])
            return 0

        lax.fori_loop(0, n_ch, chunk_body, 0)

    return k


def kernel(x, table):
    pe_flat = jnp.asarray(_pos_encoding_t_np(_LENGTH, _DEPTH).reshape(-1))
    xf = x.reshape(-1).astype(jnp.int32)
    tt = table.T
    tail2 = table[_VOCAB - 64:].reshape(32, 128)
    k1 = _make_relayout_kernel()
    t2 = k1(tt, tail2)
    k2 = _make_gather_kernel()
    out_t = k2(xf, t2, pe_flat)
    return out_t.transpose(0, 2, 1)


# speedup vs baseline: 1.0025x; 1.0025x over previous
import functools

import jax
import jax.numpy as jnp
import numpy as np
from jax import lax
from jax.experimental import pallas as pl
from jax.experimental.pallas import tpu as pltpu
from jax.experimental.pallas import tpu_sc as plsc

_VOCAB = 1000000
_DEPTH = 64
_LENGTH = 512
_BATCH = 1024

_LANES = 16
_VBLK = 128
_CH = 256


def _pos_encoding_t_np(length, depth):
    pos = np.arange(length)[:, None]
    i = np.arange(depth)[None, :]
    angle_rates = 1.0 / np.power(10000, 2 * (i // 2) / np.float32(depth))
    angle_rads = pos * angle_rates
    angle_rads[:, 0::2] = np.sin(angle_rads[:, 0::2])
    angle_rads[:, 1::2] = np.cos(angle_rads[:, 1::2])
    return np.ascontiguousarray(angle_rads.astype(np.float32).T)


def _make_relayout_kernel():
    info = plsc.get_sparse_core_info()
    nc, ns = info.num_cores, info.num_subcores
    nw = nc * ns
    n_blk = _VOCAB // _VBLK
    tail_v = _VOCAB - n_blk * _VBLK
    mesh = plsc.VectorSubcoreMesh(core_axis_name="c", subcore_axis_name="s")

    @functools.partial(
        pl.kernel,
        out_type=jax.ShapeDtypeStruct((_VOCAB // 2, 2 * _DEPTH), jnp.float32),
        mesh=mesh,
        scratch_types=[
            pltpu.VMEM((2, _DEPTH, _VBLK), jnp.float32),
            pltpu.VMEM((2, _DEPTH, _VBLK), jnp.float32),
            pltpu.VMEM((tail_v // 2, 2 * _DEPTH), jnp.float32),
            pltpu.SemaphoreType.DMA,
            pltpu.SemaphoreType.DMA,
            pltpu.SemaphoreType.DMA,
            pltpu.SemaphoreType.DMA,
        ],
        compiler_params=pltpu.CompilerParams(use_tc_tiling_on_sc=True, needs_layout_passes=False),
    )
    def k(tt_hbm, tail2_hbm, t2_hbm, slab_v, tout_v, tail_v_buf,
          in_sem0, in_sem1, out_sem0, out_sem1):
        wid = lax.axis_index("s") * nc + lax.axis_index("c")
        in_sems = [in_sem0, in_sem1]
        out_sems = [out_sem0, out_sem1]
        iota = lax.iota(jnp.int32, _LANES)
        half_iota = lax.shift_right_logical(iota, 1)
        parity64 = lax.shift_left(lax.bitwise_and(iota, 1), 6)

        n_mine = (n_blk - 1 - wid) // nw + 1

        def vb_of(i):
            return wid + i * nw

        def start_in(i, slot):
            vb = vb_of(i)
            return pltpu.async_copy(
                tt_hbm.at[:, pl.ds(vb * _VBLK, _VBLK)],
                slab_v.at[slot], in_sems[slot])

        def transpose_slab(slot):
            def body_vs(vs0, _):
                row16 = vs0 * 8 + half_iota
                col16 = parity64
                for d in range(_DEPTH):
                    vals = slab_v[slot, d, pl.ds(vs0 * _LANES, _LANES)]
                    plsc.store_scatter(tout_v.at[slot], [row16, col16], vals)
                    col16 = col16 + 1
                return 0

            lax.fori_loop(0, _VBLK // _LANES, body_vs, 0)

        @pl.when(n_mine > 0)
        def _():
            start_in(0, 0)

        def step(i, slot):
            @pl.when(i + 1 < n_mine)
            def _():
                start_in(i + 1, 1 - slot)

            pltpu.make_async_copy(
                tt_hbm.at[:, pl.ds(0, _VBLK)], slab_v.at[slot],
                in_sems[slot]).wait()

            @pl.when(i >= 2)
            def _():
                pltpu.make_async_copy(
                    tout_v.at[slot],
                    t2_hbm.at[pl.ds(0, _VBLK // 2)],
                    out_sems[slot]).wait()

            transpose_slab(slot)
            vb = vb_of(i)
            pltpu.async_copy(
                tout_v.at[slot],
                t2_hbm.at[pl.ds(vb * (_VBLK // 2), _VBLK // 2)],
                out_sems[slot])

        def pair_body(p, _):
            for sl in (0, 1):
                i = p * 2 + sl

                @pl.when(i < n_mine)
                def _():
                    step(i, sl)
            return 0

        lax.fori_loop(0, (n_mine + 1) // 2, pair_body, 0)

        for sl in (0, 1):
            @pl.when(n_mine > sl)
            def _():
                pltpu.make_async_copy(
                    tout_v.at[sl],
                    t2_hbm.at[pl.ds(0, _VBLK // 2)],
                    out_sems[sl]).wait()

        @pl.when(wid == 0)
        def _():
            pltpu.sync_copy(tail2_hbm, tail_v_buf)
            pltpu.sync_copy(
                tail_v_buf,
                t2_hbm.at[pl.ds(n_blk * (_VBLK // 2), tail_v // 2)])

    return k


def _make_gather_kernel():
    info = plsc.get_sparse_core_info()
    nc, ns = info.num_cores, info.num_subcores
    nw = nc * ns
    n_tok = _BATCH * _LENGTH
    per_w = n_tok // nw
    n_ch = per_w // _CH
    ch_per_seq = _LENGTH // _CH
    mesh = plsc.VectorSubcoreMesh(core_axis_name="c", subcore_axis_name="s")

    @functools.partial(
        pl.kernel,
        out_type=jax.ShapeDtypeStruct((_BATCH, _DEPTH, _LENGTH), jnp.float32),
        mesh=mesh,
        scratch_types=[
            pltpu.VMEM((_CH,), jnp.int32),
            pltpu.VMEM((_CH,), jnp.int32),
            pltpu.VMEM((_CH,), jnp.int32),
            pltpu.VMEM((_CH, 2 * _DEPTH), jnp.float32),
            pltpu.VMEM((_DEPTH, _CH), jnp.float32),
            pltpu.VMEM((_DEPTH * _LENGTH,), jnp.float32),
            pltpu.SemaphoreType.DMA,
        ],
        compiler_params=pltpu.CompilerParams(use_tc_tiling_on_sc=True, needs_layout_passes=False),
    )
    def k(x_hbm, t2_hbm, pe_hbm, out_hbm, idx_v, q_v, pcol_v, wide_v,
          outb_v, pe_v, sem):
        wid = lax.axis_index("s") * nc + lax.axis_index("c")
        base_w = wid * per_w
        pltpu.sync_copy(pe_hbm, pe_v)
        iota = lax.iota(jnp.int32, _LANES)

        def chunk_body(s, _):
            base = base_w + s * _CH
            b = lax.div(base, _LENGTH)
            half = lax.rem(s, ch_per_seq)
            l_off = half * _CH

            pltpu.sync_copy(x_hbm.at[pl.ds(base, _CH)], idx_v)

            def prep(i, _):
                v = idx_v[pl.ds(i * _LANES, _LANES)]
                q_v[pl.ds(i * _LANES, _LANES)] = lax.shift_right_logical(v, 1)
                pcol_v[pl.ds(i * _LANES, _LANES)] = lax.shift_left(
                    lax.bitwise_and(v, 1), 6)
                return 0

            lax.fori_loop(0, _CH // _LANES, prep, 0, unroll=4)

            pltpu.async_copy(t2_hbm.at[q_v], wide_v, sem).wait()

            def body_lb(lb, _):
                row16 = lb * _LANES + iota
                lb16 = lb * _LANES
                p_off = l_off + lb16
                col16 = pcol_v[pl.ds(lb16, _LANES)]
                for d in range(_DEPTH):
                    g = plsc.load_gather(wide_v, [row16, col16])
                    pe16 = pe_v[pl.ds(d * _LENGTH + p_off, _LANES)]
                    outb_v[d, pl.ds(lb16, _LANES)] = g + pe16
                    col16 = col16 + 1
                return 0

            lax.fori_loop(0, _CH // _LANES, body_lb, 0)

            pltpu.sync_copy(outb_v, out_hbm.at[b, :, pl.ds(l_off, _CH)])
            return 0

        lax.fori_loop(0, n_ch, chunk_body, 0)

    return k


def kernel(x, table):
    pe_flat = jnp.asarray(_pos_encoding_t_np(_LENGTH, _DEPTH).reshape(-1))
    xf = x.reshape(-1).astype(jnp.int32)
    tt = table.T
    tail2 = table[_VOCAB - 64:].reshape(32, 128)
    k1 = _make_relayout_kernel()
    t2 = k1(tt, tail2)
    k2 = _make_gather_kernel()
    out_t = k2(xf, t2, pe_flat)
    return out_t.transpose(0, 2, 1)


# speedup vs baseline: 1.0638x; 1.0612x over previous
import functools

import jax
import jax.numpy as jnp
import numpy as np
from jax import lax
from jax.experimental import pallas as pl
from jax.experimental.pallas import tpu as pltpu
from jax.experimental.pallas import tpu_sc as plsc

_VOCAB = 1000000
_DEPTH = 64
_LENGTH = 512
_BATCH = 1024

_LANES = 16
_VBLK = 128
_CH = 128


def _pos_encoding_t_np(length, depth):
    pos = np.arange(length)[:, None]
    i = np.arange(depth)[None, :]
    angle_rates = 1.0 / np.power(10000, 2 * (i // 2) / np.float32(depth))
    angle_rads = pos * angle_rates
    angle_rads[:, 0::2] = np.sin(angle_rads[:, 0::2])
    angle_rads[:, 1::2] = np.cos(angle_rads[:, 1::2])
    return np.ascontiguousarray(angle_rads.astype(np.float32).T)


def _make_relayout_kernel():
    info = plsc.get_sparse_core_info()
    nc, ns = info.num_cores, info.num_subcores
    nw = nc * ns
    n_blk = _VOCAB // _VBLK
    tail_v = _VOCAB - n_blk * _VBLK
    mesh = plsc.VectorSubcoreMesh(core_axis_name="c", subcore_axis_name="s")

    @functools.partial(
        pl.kernel,
        out_type=jax.ShapeDtypeStruct((_VOCAB // 2, 2 * _DEPTH), jnp.float32),
        mesh=mesh,
        scratch_types=[
            pltpu.VMEM((2, _DEPTH, _VBLK), jnp.float32),
            pltpu.VMEM((2, _DEPTH, _VBLK), jnp.float32),
            pltpu.VMEM((tail_v // 2, 2 * _DEPTH), jnp.float32),
            pltpu.SemaphoreType.DMA,
            pltpu.SemaphoreType.DMA,
            pltpu.SemaphoreType.DMA,
            pltpu.SemaphoreType.DMA,
        ],
        compiler_params=pltpu.CompilerParams(use_tc_tiling_on_sc=True, needs_layout_passes=False, disable_bounds_checks=True),
    )
    def k(tt_hbm, tail2_hbm, t2_hbm, slab_v, tout_v, tail_v_buf,
          in_sem0, in_sem1, out_sem0, out_sem1):
        wid = lax.axis_index("s") * nc + lax.axis_index("c")
        in_sems = [in_sem0, in_sem1]
        out_sems = [out_sem0, out_sem1]
        iota = lax.iota(jnp.int32, _LANES)
        half_iota = lax.shift_right_logical(iota, 1)
        parity64 = lax.shift_left(lax.bitwise_and(iota, 1), 6)

        n_mine = (n_blk - 1 - wid) // nw + 1

        def vb_of(i):
            return wid + i * nw

        def start_in(i, slot):
            vb = vb_of(i)
            return pltpu.async_copy(
                tt_hbm.at[:, pl.ds(vb * _VBLK, _VBLK)],
                slab_v.at[slot], in_sems[slot])

        def transpose_slab(slot):
            def body_vs(vs0, _):
                row16 = vs0 * 8 + half_iota
                col16 = parity64
                for d in range(_DEPTH):
                    vals = slab_v[slot, d, pl.ds(vs0 * _LANES, _LANES)]
                    plsc.store_scatter(tout_v.at[slot], [row16, col16], vals)
                    col16 = col16 + 1
                return 0

            lax.fori_loop(0, _VBLK // _LANES, body_vs, 0)

        @pl.when(n_mine > 0)
        def _():
            start_in(0, 0)

        def step(i, slot):
            @pl.when(i + 1 < n_mine)
            def _():
                start_in(i + 1, 1 - slot)

            pltpu.make_async_copy(
                tt_hbm.at[:, pl.ds(0, _VBLK)], slab_v.at[slot],
                in_sems[slot]).wait()

            @pl.when(i >= 2)
            def _():
                pltpu.make_async_copy(
                    tout_v.at[slot],
                    t2_hbm.at[pl.ds(0, _VBLK // 2)],
                    out_sems[slot]).wait()

            transpose_slab(slot)
            vb = vb_of(i)
            pltpu.async_copy(
                tout_v.at[slot],
                t2_hbm.at[pl.ds(vb * (_VBLK // 2), _VBLK // 2)],
                out_sems[slot])

        def pair_body(p, _):
            for sl in (0, 1):
                i = p * 2 + sl

                @pl.when(i < n_mine)
                def _():
                    step(i, sl)
            return 0

        lax.fori_loop(0, (n_mine + 1) // 2, pair_body, 0)

        for sl in (0, 1):
            @pl.when(n_mine > sl)
            def _():
                pltpu.make_async_copy(
                    tout_v.at[sl],
                    t2_hbm.at[pl.ds(0, _VBLK // 2)],
                    out_sems[sl]).wait()

        @pl.when(wid == 0)
        def _():
            pltpu.sync_copy(tail2_hbm, tail_v_buf)
            pltpu.sync_copy(
                tail_v_buf,
                t2_hbm.at[pl.ds(n_blk * (_VBLK // 2), tail_v // 2)])

    return k


def _make_gather_kernel():
    info = plsc.get_sparse_core_info()
    nc, ns = info.num_cores, info.num_subcores
    nw = nc * ns
    n_tok = _BATCH * _LENGTH
    per_w = n_tok // nw
    n_ch = per_w // _CH
    ch_per_seq = _LENGTH // _CH
    mesh = plsc.VectorSubcoreMesh(core_axis_name="c", subcore_axis_name="s")

    @functools.partial(
        pl.kernel,
        out_type=jax.ShapeDtypeStruct((_BATCH, _DEPTH, _LENGTH), jnp.float32),
        mesh=mesh,
        scratch_types=[
            pltpu.VMEM((2, _CH), jnp.int32),
            pltpu.VMEM((2, _CH), jnp.int32),
            pltpu.VMEM((2, _CH), jnp.int32),
            pltpu.VMEM((2, _CH, 2 * _DEPTH), jnp.float32),
            pltpu.VMEM((2, _DEPTH, _CH), jnp.float32),
            pltpu.VMEM((_DEPTH * _LENGTH,), jnp.float32),
            pltpu.SemaphoreType.DMA,
            pltpu.SemaphoreType.DMA,
            pltpu.SemaphoreType.DMA,
            pltpu.SemaphoreType.DMA,
        ],
        compiler_params=pltpu.CompilerParams(use_tc_tiling_on_sc=True, needs_layout_passes=False, disable_bounds_checks=True),
    )
    def k(x_hbm, t2_hbm, pe_hbm, out_hbm, idx_v, q_v, pcol_v, wide_v,
          outb_v, pe_v, g_sem0, g_sem1, o_sem0, o_sem1):
        wid = lax.axis_index("s") * nc + lax.axis_index("c")
        base_w = wid * per_w
        pltpu.sync_copy(pe_hbm, pe_v)
        iota = lax.iota(jnp.int32, _LANES)
        g_sems = [g_sem0, g_sem1]
        o_sems = [o_sem0, o_sem1]

        def start_gather(s, slot):
            base = base_w + s * _CH
            pltpu.sync_copy(x_hbm.at[pl.ds(base, _CH)], idx_v.at[slot])

            def prep(i, _):
                v = idx_v[slot, pl.ds(i * _LANES, _LANES)]
                q_v[slot, pl.ds(i * _LANES, _LANES)] = (
                    lax.shift_right_logical(v, 1))
                pcol_v[slot, pl.ds(i * _LANES, _LANES)] = lax.shift_left(
                    lax.bitwise_and(v, 1), 6)
                return 0

            lax.fori_loop(0, _CH // _LANES, prep, 0)
            pltpu.async_copy(t2_hbm.at[q_v.at[slot]], wide_v.at[slot],
                             g_sems[slot])

        def step(s, slot):
            @pl.when(s + 1 < n_ch)
            def _():
                start_gather(s + 1, 1 - slot)

            pltpu.make_async_copy(
                t2_hbm.at[pl.ds(0, _CH)], wide_v.at[slot],
                g_sems[slot]).wait()

            @pl.when(s >= 2)
            def _():
                pltpu.make_async_copy(
                    outb_v.at[slot],
                    out_hbm.at[0, :, pl.ds(0, _CH)],
                    o_sems[slot]).wait()

            base = base_w + s * _CH
            b = lax.div(base, _LENGTH)
            half = lax.rem(s, ch_per_seq)
            l_off = half * _CH

            def body_lb(lb, _):
                row16 = lb * _LANES + iota
                lb16 = lb * _LANES
                p_off = l_off + lb16
                p16 = pcol_v[slot, pl.ds(lb16, _LANES)]
                for d in range(_DEPTH):
                    g = plsc.load_gather(wide_v.at[slot], [row16, p16 + d])
                    pe16 = pe_v[pl.ds(d * _LENGTH + p_off, _LANES)]
                    outb_v[slot, d, pl.ds(lb16, _LANES)] = g + pe16
                return 0

            lax.fori_loop(0, _CH // _LANES, body_lb, 0)

            pltpu.async_copy(outb_v.at[slot],
                             out_hbm.at[b, :, pl.ds(l_off, _CH)],
                             o_sems[slot])

        start_gather(0, 0)

        def pair_body(p, _):
            for sl in (0, 1):
                s = p * 2 + sl

                @pl.when(s < n_ch)
                def _():
                    step(s, sl)
            return 0

        lax.fori_loop(0, (n_ch + 1) // 2, pair_body, 0)

        for sl in (0, 1):
            @pl.when(n_ch > sl)
            def _():
                pltpu.make_async_copy(
                    outb_v.at[sl],
                    out_hbm.at[0, :, pl.ds(0, _CH)],
                    o_sems[sl]).wait()

    return k


def kernel(x, table):
    pe_flat = jnp.asarray(_pos_encoding_t_np(_LENGTH, _DEPTH).reshape(-1))
    xf = x.reshape(-1).astype(jnp.int32)
    tt = table.T
    tail2 = table[_VOCAB - 64:].reshape(32, 128)
    k1 = _make_relayout_kernel()
    t2 = k1(tt, tail2)
    k2 = _make_gather_kernel()
    out_t = k2(xf, t2, pe_flat)
    return out_t.transpose(0, 2, 1)
